# baseline (speedup 1.0000x reference)
import functools

import jax
import jax.numpy as jnp
from jax.experimental import pallas as pl
from jax.experimental.pallas import tpu as pltpu

_F = 2048
_G = 32
_D = 64
_PACK = 4
_S = _G // _PACK
_SW = _PACK * _D
_NS_ITERS = 12


def _dot(a, b, dims):
    return jax.lax.dot_general(a, b, (dims, ((), ())),
                               preferred_element_type=jnp.float32)


def _whiten_slab(g, cf_slab, mu, n, m_ref):
    ri = jax.lax.broadcasted_iota(jnp.int32, (_F, _SW), 0)
    p_s = (ri == cf_slab).astype(jnp.float32)

    gp = _dot(g, p_s, ((1,), (0,)))
    gu = _dot(p_s, gp, ((0,), (0,)))
    gd = jnp.zeros((_SW, _SW), jnp.float32)
    for b in range(_S):
        pb = p_s[b * _SW:(b + 1) * _SW, :]
        db = g[b * _SW:(b + 1) * _SW, b * _SW:(b + 1) * _SW]
        gd = gd + _dot(pb, _dot(db, pb, ((1,), (0,))), ((0,), (0,)))
    gs = gu + gu.T - gd

    mu_s = _dot(mu, p_s, ((1,), (0,)))
    outer = _dot(mu_s, mu_s, ((0,), (0,)))

    ri2 = jax.lax.broadcasted_iota(jnp.int32, (_SW, _SW), 0)
    ci2 = jax.lax.broadcasted_iota(jnp.int32, (_SW, _SW), 1)
    mask = ((ri2 // _D) == (ci2 // _D)).astype(jnp.float32)
    eye = (ri2 == ci2).astype(jnp.float32)

    cov = (gs - n * outer) * mask * (1.0 / _G)
    rowsum = jnp.sum(jnp.abs(cov), axis=-1, keepdims=True)
    s = jnp.maximum(jnp.max(rowsum), 1e-30)
    a = cov * (1.0 / s)

    y = a
    z = eye
    for _ in range(_NS_ITERS):
        zy = _dot(z, y, ((1,), (0,)))
        t = 1.5 * eye - 0.5 * zy
        y = _dot(y, t, ((1,), (0,)))
        z = _dot(t, z, ((1,), (0,)))
    w_s = z * jax.lax.rsqrt(s)

    pw = _dot(p_s, w_s, ((1,), (0,)))
    q = _F // 4
    for j in range(4):
        mcq = _dot(pw, p_s[j * q:(j + 1) * q, :], ((1,), (1,)))
        m_ref[:, j * q:(j + 1) * q] += mcq


def _stats_whiten_kernel(x_ref, cf_ref, mean_ref, m_ref, gram_scr,
                         *, inv_n, n, n_blocks):
    k = pl.program_id(0)

    @pl.when(k == 0)
    def _init():
        mean_ref[...] = jnp.zeros_like(mean_ref)
        gram_scr[...] = jnp.zeros_like(gram_scr)

    @pl.when(k < n_blocks)
    def _stats():
        xb = x_ref[...]
        mean_ref[...] += jnp.sum(xb, axis=0, keepdims=True)
        for bi in range(_S):
            xi = xb[:, bi * _SW:(bi + 1) * _SW]
            for bj in range(bi, _S):
                xj = xb[:, bj * _SW:(bj + 1) * _SW]
                gram_scr[bi * _SW:(bi + 1) * _SW,
                         bj * _SW:(bj + 1) * _SW] += (
                    jax.lax.dot_general(xi, xj, (((0,), (0,)), ((), ())),
                                        preferred_element_type=jnp.float32))

    @pl.when(k == n_blocks)
    def _whiten():
        mean_ref[...] *= inv_n
        m_ref[...] = jnp.zeros_like(m_ref)
        g = gram_scr[...]
        mu = mean_ref[...]
        for s in range(_S):
            _whiten_slab(g, cf_ref[0:1, s * _SW:(s + 1) * _SW], mu, n, m_ref)


def _apply_kernel(x_ref, m_ref, mu_ref, o_ref):
    xc = x_ref[...] - mu_ref[...]
    o_ref[...] = jax.lax.dot_general(
        xc, m_ref[...], (((1,), (0,)), ((), ())),
        preferred_element_type=jnp.float32)


def kernel(x, shuffle_idx):
    n_rows, f = x.shape
    assert f == _F
    cf = shuffle_idx.astype(jnp.int32).reshape(1, _F)

    blk = 1024
    nb = n_rows // blk

    mean, m = pl.pallas_call(
        functools.partial(_stats_whiten_kernel, inv_n=1.0 / n_rows,
                          n=float(n_rows), n_blocks=nb),
        grid=(nb + 1,),
        in_specs=[
            pl.BlockSpec((blk, _F), lambda k: (jnp.minimum(k, nb - 1), 0)),
            pl.BlockSpec((1, _F), lambda k: (0, 0)),
        ],
        out_specs=[
            pl.BlockSpec((1, _F), lambda k: (0, 0)),
            pl.BlockSpec((_F, _F), lambda k: (0, 0)),
        ],
        out_shape=[
            jax.ShapeDtypeStruct((1, _F), jnp.float32),
            jax.ShapeDtypeStruct((_F, _F), jnp.float32),
        ],
        scratch_shapes=[pltpu.VMEM((_F, _F), jnp.float32)],
        compiler_params=pltpu.CompilerParams(
            dimension_semantics=("arbitrary",)),
    )(x, cf)

    out = pl.pallas_call(
        _apply_kernel,
        grid=(nb,),
        in_specs=[
            pl.BlockSpec((blk, _F), lambda k: (k, 0)),
            pl.BlockSpec((_F, _F), lambda k: (0, 0)),
            pl.BlockSpec((1, _F), lambda k: (0, 0)),
        ],
        out_specs=pl.BlockSpec((blk, _F), lambda k: (k, 0)),
        out_shape=jax.ShapeDtypeStruct((n_rows, _F), jnp.float32),
        compiler_params=pltpu.CompilerParams(
            dimension_semantics=("arbitrary",)),
    )(x, m, mean)

    return out


# speedup vs baseline: 1.0926x; 1.0926x over previous
import functools

import jax
import jax.numpy as jnp
from jax.experimental import pallas as pl
from jax.experimental.pallas import tpu as pltpu

_F = 2048
_G = 32
_D = 64
_PACK = 4
_S = _G // _PACK
_SW = _PACK * _D
_NS_ITERS = 12


def _stats_kernel(x_ref, mean_ref, gram_ref, *, inv_n, last_k):
    k = pl.program_id(0)

    @pl.when(k == 0)
    def _init():
        mean_ref[...] = jnp.zeros_like(mean_ref)
        gram_ref[...] = jnp.zeros_like(gram_ref)

    xb = x_ref[...]
    mean_ref[...] += jnp.sum(xb, axis=0, keepdims=True)
    for bi in range(_S):
        xi = xb[:, bi * _SW:(bi + 1) * _SW]
        for bj in range(bi, _S):
            xj = xb[:, bj * _SW:(bj + 1) * _SW]
            gram_ref[bi * _SW:(bi + 1) * _SW, bj * _SW:(bj + 1) * _SW] += (
                jax.lax.dot_general(xi, xj, (((0,), (0,)), ((), ())),
                                    preferred_element_type=jnp.float32))

    @pl.when(k == last_k)
    def _finish():
        mean_ref[...] *= inv_n
        for b in range(_S):
            gram_ref[b * _SW:(b + 1) * _SW, b * _SW:(b + 1) * _SW] *= 0.5


def _dot(a, b, dims):
    return jax.lax.dot_general(a, b, (dims, ((), ())),
                               preferred_element_type=jnp.float32)


def _whiten_kernel(g_ref, cf_ref, mu_ref, m_ref, *, n):
    k = pl.program_id(0)

    ri = jax.lax.broadcasted_iota(jnp.int32, (_F, _SW), 0)
    p_s = (ri == cf_ref[0]).astype(jnp.float32)

    gp = _dot(g_ref[...], p_s, ((1,), (0,)))
    gu = _dot(p_s, gp, ((0,), (0,)))
    gs = gu + gu.T
    mu_s = _dot(mu_ref[...], p_s, ((1,), (0,)))
    outer = _dot(mu_s, mu_s, ((0,), (0,)))

    ri2 = jax.lax.broadcasted_iota(jnp.int32, (_SW, _SW), 0)
    ci2 = jax.lax.broadcasted_iota(jnp.int32, (_SW, _SW), 1)
    mask = ((ri2 // _D) == (ci2 // _D)).astype(jnp.float32)
    eye = (ri2 == ci2).astype(jnp.float32)

    cov = (gs - n * outer) * mask * (1.0 / _G)
    rowsum = jnp.sum(jnp.abs(cov), axis=-1, keepdims=True)
    s = jnp.maximum(jnp.max(rowsum), 1e-30)
    a = cov * (1.0 / s)

    y = a
    z = eye
    for _ in range(_NS_ITERS):
        zy = _dot(z, y, ((1,), (0,)))
        t = 1.5 * eye - 0.5 * zy
        y = _dot(y, t, ((1,), (0,)))
        z = _dot(t, z, ((1,), (0,)))
    w_s = z * jax.lax.rsqrt(s)

    pw = _dot(p_s, w_s, ((1,), (0,)))

    @pl.when(k == 0)
    def _init():
        m_ref[...] = jnp.zeros_like(m_ref)

    q = _F // 4
    for j in range(4):
        mcq = _dot(pw, p_s[j * q:(j + 1) * q, :], ((1,), (1,)))
        m_ref[:, j * q:(j + 1) * q] += mcq


def _apply_kernel(x_ref, m_ref, mu_ref, o_ref):
    xc = x_ref[...] - mu_ref[...]
    o_ref[...] = jax.lax.dot_general(
        xc, m_ref[...], (((1,), (0,)), ((), ())),
        preferred_element_type=jnp.float32)


def kernel(x, shuffle_idx):
    n_rows, f = x.shape
    assert f == _F
    cf = shuffle_idx.astype(jnp.int32)
    cf3 = cf.reshape(_S, 1, _SW)

    blk = 1024
    blk_stats = 1024

    mean, gram = pl.pallas_call(
        functools.partial(_stats_kernel, inv_n=1.0 / n_rows,
                          last_k=n_rows // blk_stats - 1),
        grid=(n_rows // blk_stats,),
        in_specs=[pl.BlockSpec((blk_stats, _F), lambda k: (k, 0))],
        out_specs=[
            pl.BlockSpec((1, _F), lambda k: (0, 0)),
            pl.BlockSpec((_F, _F), lambda k: (0, 0)),
        ],
        out_shape=[
            jax.ShapeDtypeStruct((1, _F), jnp.float32),
            jax.ShapeDtypeStruct((_F, _F), jnp.float32),
        ],
        compiler_params=pltpu.CompilerParams(
            dimension_semantics=("arbitrary",)),
    )(x)

    m = pl.pallas_call(
        functools.partial(_whiten_kernel, n=float(n_rows)),
        grid=(_S,),
        in_specs=[
            pl.BlockSpec((_F, _F), lambda k: (0, 0)),
            pl.BlockSpec((1, 1, _SW), lambda k: (k, 0, 0)),
            pl.BlockSpec((1, _F), lambda k: (0, 0)),
        ],
        out_specs=pl.BlockSpec((_F, _F), lambda k: (0, 0)),
        out_shape=jax.ShapeDtypeStruct((_F, _F), jnp.float32),
        compiler_params=pltpu.CompilerParams(
            dimension_semantics=("arbitrary",)),
    )(gram, cf3, mean)

    out = pl.pallas_call(
        _apply_kernel,
        grid=(n_rows // blk,),
        in_specs=[
            pl.BlockSpec((blk, _F), lambda k: (k, 0)),
            pl.BlockSpec((_F, _F), lambda k: (0, 0)),
            pl.BlockSpec((1, _F), lambda k: (0, 0)),
        ],
        out_specs=pl.BlockSpec((blk, _F), lambda k: (k, 0)),
        out_shape=jax.ShapeDtypeStruct((n_rows, _F), jnp.float32),
        compiler_params=pltpu.CompilerParams(
            dimension_semantics=("arbitrary",)),
    )(x, m, mean)

    return out


# speedup vs baseline: 1.1177x; 1.0230x over previous
import functools

import jax
import jax.numpy as jnp
from jax.experimental import pallas as pl
from jax.experimental.pallas import tpu as pltpu

_F = 2048
_G = 32
_D = 64
_PACK = 4
_S = _G // _PACK
_SW = _PACK * _D
_NS_ITERS = 10


def _stats_kernel(x_ref, mean_ref, gram_ref, *, inv_n, last_k):
    k = pl.program_id(0)

    @pl.when(k == 0)
    def _init():
        mean_ref[...] = jnp.zeros_like(mean_ref)
        gram_ref[...] = jnp.zeros_like(gram_ref)

    xb = x_ref[...]
    mean_ref[...] += jnp.sum(xb, axis=0, keepdims=True)
    for bi in range(_S):
        xi = xb[:, bi * _SW:(bi + 1) * _SW]
        for bj in range(bi, _S):
            xj = xb[:, bj * _SW:(bj + 1) * _SW]
            gram_ref[bi * _SW:(bi + 1) * _SW, bj * _SW:(bj + 1) * _SW] += (
                jax.lax.dot_general(xi, xj, (((0,), (0,)), ((), ())),
                                    preferred_element_type=jnp.float32))

    @pl.when(k == last_k)
    def _finish():
        mean_ref[...] *= inv_n
        for b in range(_S):
            gram_ref[b * _SW:(b + 1) * _SW, b * _SW:(b + 1) * _SW] *= 0.5


def _dot(a, b, dims):
    return jax.lax.dot_general(a, b, (dims, ((), ())),
                               preferred_element_type=jnp.float32)


def _whiten_kernel(g_ref, cf_ref, cff_ref, mu_ref, m_ref, gp_scr, *, n):
    k = pl.program_id(0)

    @pl.when(k == 0)
    def _gp_once():
        qw = _F // 4
        riq = jax.lax.broadcasted_iota(jnp.int32, (_F, qw), 0)
        for qi in range(4):
            pq = (riq == cff_ref[0:1, qi * qw:(qi + 1) * qw]).astype(
                jnp.float32)
            gp_scr[:, qi * qw:(qi + 1) * qw] = _dot(
                g_ref[...], pq, ((1,), (0,)))

    ri = jax.lax.broadcasted_iota(jnp.int32, (_F, _SW), 0)
    p_s = (ri == cf_ref[0]).astype(jnp.float32)

    off = pl.multiple_of(k * _SW, _SW)
    gp = gp_scr[:, pl.ds(off, _SW)]
    gu = _dot(p_s, gp, ((0,), (0,)))
    gs = gu + gu.T
    mu_s = _dot(mu_ref[...], p_s, ((1,), (0,)))
    outer = _dot(mu_s, mu_s, ((0,), (0,)))

    ri2 = jax.lax.broadcasted_iota(jnp.int32, (_SW, _SW), 0)
    ci2 = jax.lax.broadcasted_iota(jnp.int32, (_SW, _SW), 1)
    mask = ((ri2 // _D) == (ci2 // _D)).astype(jnp.float32)
    eye = (ri2 == ci2).astype(jnp.float32)

    cov = (gs - n * outer) * mask * (1.0 / _G)
    rowsum = jnp.sum(jnp.abs(cov), axis=-1, keepdims=True)
    s = jnp.maximum(jnp.max(rowsum), 1e-30)
    a = cov * (1.0 / s)

    y = a
    z = eye
    for _ in range(_NS_ITERS):
        zy = _dot(z, y, ((1,), (0,)))
        t = 1.5 * eye - 0.5 * zy
        y = _dot(y, t, ((1,), (0,)))
        z = _dot(t, z, ((1,), (0,)))
    w_s = z * jax.lax.rsqrt(s)

    pw = _dot(p_s, w_s, ((1,), (0,)))

    @pl.when(k == 0)
    def _init():
        m_ref[...] = jnp.zeros_like(m_ref)

    q = _F // 4
    for j in range(4):
        mcq = _dot(pw, p_s[j * q:(j + 1) * q, :], ((1,), (1,)))
        m_ref[:, j * q:(j + 1) * q] += mcq


def _apply_kernel(x_ref, m_ref, mu_ref, o_ref):
    xc = x_ref[...] - mu_ref[...]
    o_ref[...] = jax.lax.dot_general(
        xc, m_ref[...], (((1,), (0,)), ((), ())),
        preferred_element_type=jnp.float32)


def kernel(x, shuffle_idx):
    n_rows, f = x.shape
    assert f == _F
    cf = shuffle_idx.astype(jnp.int32)
    cf3 = cf.reshape(_S, 1, _SW)

    blk = 1024
    blk_stats = 1024

    mean, gram = pl.pallas_call(
        functools.partial(_stats_kernel, inv_n=1.0 / n_rows,
                          last_k=n_rows // blk_stats - 1),
        grid=(n_rows // blk_stats,),
        in_specs=[pl.BlockSpec((blk_stats, _F), lambda k: (k, 0))],
        out_specs=[
            pl.BlockSpec((1, _F), lambda k: (0, 0)),
            pl.BlockSpec((_F, _F), lambda k: (0, 0)),
        ],
        out_shape=[
            jax.ShapeDtypeStruct((1, _F), jnp.float32),
            jax.ShapeDtypeStruct((_F, _F), jnp.float32),
        ],
        compiler_params=pltpu.CompilerParams(
            dimension_semantics=("arbitrary",)),
    )(x)

    m = pl.pallas_call(
        functools.partial(_whiten_kernel, n=float(n_rows)),
        grid=(_S,),
        in_specs=[
            pl.BlockSpec((_F, _F), lambda k: (0, 0)),
            pl.BlockSpec((1, 1, _SW), lambda k: (k, 0, 0)),
            pl.BlockSpec((1, _F), lambda k: (0, 0)),
            pl.BlockSpec((1, _F), lambda k: (0, 0)),
        ],
        out_specs=pl.BlockSpec((_F, _F), lambda k: (0, 0)),
        out_shape=jax.ShapeDtypeStruct((_F, _F), jnp.float32),
        scratch_shapes=[pltpu.VMEM((_F, _F), jnp.float32)],
        compiler_params=pltpu.CompilerParams(
            dimension_semantics=("arbitrary",)),
    )(gram, cf3, cf.reshape(1, _F), mean)

    out = pl.pallas_call(
        _apply_kernel,
        grid=(n_rows // blk,),
        in_specs=[
            pl.BlockSpec((blk, _F), lambda k: (k, 0)),
            pl.BlockSpec((_F, _F), lambda k: (0, 0)),
            pl.BlockSpec((1, _F), lambda k: (0, 0)),
        ],
        out_specs=pl.BlockSpec((blk, _F), lambda k: (k, 0)),
        out_shape=jax.ShapeDtypeStruct((n_rows, _F), jnp.float32),
        compiler_params=pltpu.CompilerParams(
            dimension_semantics=("arbitrary",)),
    )(x, m, mean)

    return out


# speedup vs baseline: 1.1443x; 1.0237x over previous
import functools

import jax
import jax.numpy as jnp
from jax.experimental import pallas as pl
from jax.experimental.pallas import tpu as pltpu

_F = 2048
_G = 32
_D = 64
_PACK = 4
_S = _G // _PACK
_SW = _PACK * _D
_NS_ITERS = 10


def _stats_kernel(x_ref, mean_ref, gram_ref, *, inv_n, last_k):
    k = pl.program_id(0)

    @pl.when(k == 0)
    def _init():
        mean_ref[...] = jnp.zeros_like(mean_ref)
        gram_ref[...] = jnp.zeros_like(gram_ref)

    xb = x_ref[...]
    mean_ref[...] += jnp.sum(xb, axis=0, keepdims=True)
    for bi in range(_S):
        xi = xb[:, bi * _SW:(bi + 1) * _SW]
        for bj in range(bi, _S):
            xj = xb[:, bj * _SW:(bj + 1) * _SW]
            gram_ref[bi * _SW:(bi + 1) * _SW, bj * _SW:(bj + 1) * _SW] += (
                jax.lax.dot_general(xi, xj, (((0,), (0,)), ((), ())),
                                    preferred_element_type=jnp.float32))

    @pl.when(k == last_k)
    def _finish():
        mean_ref[...] *= inv_n
        for b in range(_S):
            gram_ref[b * _SW:(b + 1) * _SW, b * _SW:(b + 1) * _SW] *= 0.5


def _dot(a, b, dims):
    return jax.lax.dot_general(a, b, (dims, ((), ())),
                               preferred_element_type=jnp.float32)


def _whiten_kernel(g_ref, cff_ref, mu_ref, m_ref, gp_scr, *, n):
    qw = _F // 4
    riq = jax.lax.broadcasted_iota(jnp.int32, (_F, qw), 0)
    for qi in range(4):
        pq = (riq == cff_ref[0:1, qi * qw:(qi + 1) * qw]).astype(
            jnp.float32)
        gp_scr[:, qi * qw:(qi + 1) * qw] = _dot(g_ref[...], pq, ((1,), (0,)))

    ri = jax.lax.broadcasted_iota(jnp.int32, (_F, _SW), 0)
    ri2 = jax.lax.broadcasted_iota(jnp.int32, (_SW, _SW), 0)
    ci2 = jax.lax.broadcasted_iota(jnp.int32, (_SW, _SW), 1)
    mask = ((ri2 // _D) == (ci2 // _D)).astype(jnp.float32)
    eye = (ri2 == ci2).astype(jnp.float32)

    m_ref[...] = jnp.zeros_like(m_ref)
    q = _F // 4

    for sl in range(_S):
        p_s = (ri == cff_ref[0:1, sl * _SW:(sl + 1) * _SW]).astype(
            jnp.float32)

        gp = gp_scr[:, sl * _SW:(sl + 1) * _SW]
        gu = _dot(p_s, gp, ((0,), (0,)))
        gs = gu + gu.T
        mu_s = _dot(mu_ref[...], p_s, ((1,), (0,)))
        outer = _dot(mu_s, mu_s, ((0,), (0,)))

        cov = (gs - n * outer) * mask * (1.0 / _G)
        rowsum = jnp.sum(jnp.abs(cov), axis=-1, keepdims=True)
        s = jnp.maximum(jnp.max(rowsum), 1e-30)
        a = cov * (1.0 / s)

        y = a
        z = eye
        for _ in range(_NS_ITERS):
            zy = _dot(z, y, ((1,), (0,)))
            t = 1.5 * eye - 0.5 * zy
            y = _dot(y, t, ((1,), (0,)))
            z = _dot(t, z, ((1,), (0,)))
        w_s = z * jax.lax.rsqrt(s)

        pw = _dot(p_s, w_s, ((1,), (0,)))
        for j in range(4):
            mcq = _dot(pw, p_s[j * q:(j + 1) * q, :], ((1,), (1,)))
            m_ref[:, j * q:(j + 1) * q] += mcq


def _apply_kernel(x_ref, m_ref, mu_ref, o_ref):
    xc = x_ref[...] - mu_ref[...]
    o_ref[...] = jax.lax.dot_general(
        xc, m_ref[...], (((1,), (0,)), ((), ())),
        preferred_element_type=jnp.float32)


def kernel(x, shuffle_idx):
    n_rows, f = x.shape
    assert f == _F
    cf = shuffle_idx.astype(jnp.int32)

    blk = 1024
    blk_stats = 1024

    mean, gram = pl.pallas_call(
        functools.partial(_stats_kernel, inv_n=1.0 / n_rows,
                          last_k=n_rows // blk_stats - 1),
        grid=(n_rows // blk_stats,),
        in_specs=[pl.BlockSpec((blk_stats, _F), lambda k: (k, 0))],
        out_specs=[
            pl.BlockSpec((1, _F), lambda k: (0, 0)),
            pl.BlockSpec((_F, _F), lambda k: (0, 0)),
        ],
        out_shape=[
            jax.ShapeDtypeStruct((1, _F), jnp.float32),
            jax.ShapeDtypeStruct((_F, _F), jnp.float32),
        ],
        compiler_params=pltpu.CompilerParams(
            dimension_semantics=("arbitrary",)),
    )(x)

    m = pl.pallas_call(
        functools.partial(_whiten_kernel, n=float(n_rows)),
        grid=(1,),
        in_specs=[
            pl.BlockSpec((_F, _F), lambda k: (0, 0)),
            pl.BlockSpec((1, _F), lambda k: (0, 0)),
            pl.BlockSpec((1, _F), lambda k: (0, 0)),
        ],
        out_specs=pl.BlockSpec((_F, _F), lambda k: (0, 0)),
        out_shape=jax.ShapeDtypeStruct((_F, _F), jnp.float32),
        scratch_shapes=[pltpu.VMEM((_F, _F), jnp.float32)],
        compiler_params=pltpu.CompilerParams(
            dimension_semantics=("arbitrary",)),
    )(gram, cf.reshape(1, _F), mean)

    out = pl.pallas_call(
        _apply_kernel,
        grid=(n_rows // blk,),
        in_specs=[
            pl.BlockSpec((blk, _F), lambda k: (k, 0)),
            pl.BlockSpec((_F, _F), lambda k: (0, 0)),
            pl.BlockSpec((1, _F), lambda k: (0, 0)),
        ],
        out_specs=pl.BlockSpec((blk, _F), lambda k: (k, 0)),
        out_shape=jax.ShapeDtypeStruct((n_rows, _F), jnp.float32),
        compiler_params=pltpu.CompilerParams(
            dimension_semantics=("arbitrary",)),
    )(x, m, mean)

    return out
